# Initial kernel scaffold; baseline (speedup 1.0000x reference)
import jax
import jax.numpy as jnp
from jax.experimental import pallas as pl


def kernel(x, emb_table, fc_w, fc_b):
    raise NotImplementedError("write your pallas kernel here")


# speedup vs baseline: 1.2787x; 1.2787x over previous
import functools

import jax
import jax.numpy as jnp
from jax import lax
from jax.experimental import pallas as pl
from jax.experimental.pallas import tpu as pltpu
from jax.experimental.pallas import tpu_sc as plsc

_VOCAB = 100000
_EMB = 32
_B = 1024
_L = 200

_NC = 2
_NS = 16
_NW = _NC * _NS
_BPW = _B // _NW
_CH = 100
_CPW = _BPW * _L // _CH
_LANES = 16

_mesh = plsc.VectorSubcoreMesh(core_axis_name="c", subcore_axis_name="s")


@functools.partial(
    pl.kernel,
    mesh=_mesh,
    out_type=jax.ShapeDtypeStruct((_B, _EMB), jnp.float32),
    scratch_types=[
        pltpu.VMEM((_CPW, _CH), jnp.int32),
        pltpu.VMEM((_CH, _EMB), jnp.float32),
        pltpu.VMEM((_CH, _EMB), jnp.float32),
        pltpu.VMEM((_BPW, _EMB), jnp.float32),
        pltpu.SemaphoreType.DMA,
        pltpu.SemaphoreType.DMA,
    ],
    compiler_params=pltpu.CompilerParams(use_tc_tiling_on_sc=False),
)
def _pool(x_hbm, table_hbm, out_hbm, idx_v, rows0_v, rows1_v, pooled_v,
          sem0, sem1):
    wid = lax.axis_index("s") * _NC + lax.axis_index("c")
    pltpu.sync_copy(x_hbm.at[pl.ds(wid * _CPW, _CPW)], idx_v)
    inv_l = jnp.float32(1.0 / _L)

    def _accum(buf):
        def body(l, c):
            a0, a1 = c
            return (a0 + buf[l, pl.ds(0, _LANES)],
                    a1 + buf[l, pl.ds(_LANES, _LANES)])
        z = jnp.zeros((_LANES,), jnp.float32)
        return lax.fori_loop(0, _CH, body, (z, z))

    def row_body(i, carry):
        j0 = 2 * i
        pltpu.async_copy(table_hbm.at[idx_v.at[j0]], rows0_v, sem0)
        pltpu.async_copy(table_hbm.at[idx_v.at[j0 + 1]], rows1_v, sem1)
        pltpu.make_async_copy(table_hbm.at[idx_v.at[j0]], rows0_v, sem0).wait()
        a0, a1 = _accum(rows0_v)
        pltpu.make_async_copy(table_hbm.at[idx_v.at[j0 + 1]], rows1_v,
                              sem1).wait()
        b0, b1 = _accum(rows1_v)
        pooled_v[i, pl.ds(0, _LANES)] = (a0 + b0) * inv_l
        pooled_v[i, pl.ds(_LANES, _LANES)] = (a1 + b1) * inv_l
        return carry

    lax.fori_loop(0, _BPW, row_body, 0)
    pltpu.sync_copy(pooled_v, out_hbm.at[pl.ds(wid * _BPW, _BPW)])


_VT = 512


def _fc_body(p_ref, w_ref, b_ref, o_ref):
    o_ref[...] = lax.dot_general(
        p_ref[...], w_ref[...],
        dimension_numbers=(((1,), (1,)), ((), ())),
        preferred_element_type=jnp.float32,
    ) + b_ref[...]


def _fc(pooled, fc_w, fc_b2):
    return pl.pallas_call(
        _fc_body,
        grid=(pl.cdiv(_VOCAB, _VT),),
        in_specs=[
            pl.BlockSpec((_B, _EMB), lambda i: (0, 0)),
            pl.BlockSpec((_VT, _EMB), lambda i: (i, 0)),
            pl.BlockSpec((1, _VT), lambda i: (0, i)),
        ],
        out_specs=pl.BlockSpec((_B, _VT), lambda i: (0, i)),
        out_shape=jax.ShapeDtypeStruct((_B, _VOCAB), jnp.float32),
    )(pooled, fc_w, fc_b2)


def kernel(x, emb_table, fc_w, fc_b):
    xi = x.astype(jnp.int32).reshape(_B * _L // _CH, _CH)
    pooled = _pool(xi, emb_table)
    return _fc(pooled, fc_w, fc_b.reshape(1, _VOCAB))


# speedup vs baseline: 1.3970x; 1.0925x over previous
import functools

import jax
import jax.numpy as jnp
from jax import lax
from jax.experimental import pallas as pl
from jax.experimental.pallas import tpu as pltpu
from jax.experimental.pallas import tpu_sc as plsc

_VOCAB = 100000
_EMB = 32
_B = 1024
_L = 200

_NC = 2
_NS = 16
_NW = _NC * _NS
_BPW = _B // _NW
_CH = 100
_CPW = _BPW * _L // _CH
_LANES = 16

_mesh = plsc.VectorSubcoreMesh(core_axis_name="c", subcore_axis_name="s")


@functools.partial(
    pl.kernel,
    mesh=_mesh,
    out_type=jax.ShapeDtypeStruct((_B, _EMB), jnp.float32),
    scratch_types=[
        pltpu.VMEM((_CPW, _CH), jnp.int32),
        pltpu.VMEM((_CH, _EMB), jnp.float32),
        pltpu.VMEM((_CH, _EMB), jnp.float32),
        pltpu.VMEM((_BPW, _EMB), jnp.float32),
        pltpu.SemaphoreType.DMA,
        pltpu.SemaphoreType.DMA,
    ],
    compiler_params=pltpu.CompilerParams(use_tc_tiling_on_sc=False),
)
def _pool(x_hbm, table_hbm, out_hbm, idx_v, rows0_v, rows1_v, pooled_v,
          sem0, sem1):
    wid = lax.axis_index("s") * _NC + lax.axis_index("c")
    pltpu.sync_copy(x_hbm.at[pl.ds(wid * _CPW, _CPW)], idx_v)
    inv_l = jnp.float32(1.0 / _L)

    def _accum(buf):
        def body(l, c):
            a0, a1 = c
            return (a0 + buf[l, pl.ds(0, _LANES)],
                    a1 + buf[l, pl.ds(_LANES, _LANES)])
        z = jnp.zeros((_LANES,), jnp.float32)
        return lax.fori_loop(0, _CH, body, (z, z))

    def row_body(i, carry):
        j0 = 2 * i
        pltpu.async_copy(table_hbm.at[idx_v.at[j0]], rows0_v, sem0)
        pltpu.async_copy(table_hbm.at[idx_v.at[j0 + 1]], rows1_v, sem1)
        pltpu.make_async_copy(table_hbm.at[idx_v.at[j0]], rows0_v, sem0).wait()
        a0, a1 = _accum(rows0_v)
        pltpu.make_async_copy(table_hbm.at[idx_v.at[j0 + 1]], rows1_v,
                              sem1).wait()
        b0, b1 = _accum(rows1_v)
        pooled_v[i, pl.ds(0, _LANES)] = (a0 + b0) * inv_l
        pooled_v[i, pl.ds(_LANES, _LANES)] = (a1 + b1) * inv_l
        return carry

    lax.fori_loop(0, _BPW, row_body, 0)
    pltpu.sync_copy(pooled_v, out_hbm.at[pl.ds(wid * _BPW, _BPW)])


_VT = 1024


def _fc_body(p_ref, w_ref, b_ref, o_ref):
    o_ref[...] = lax.dot_general(
        p_ref[...], w_ref[...],
        dimension_numbers=(((1,), (1,)), ((), ())),
        preferred_element_type=jnp.float32,
    ) + b_ref[...]


def _fc(pooled, fc_w, fc_b2):
    return pl.pallas_call(
        _fc_body,
        grid=(pl.cdiv(_VOCAB, _VT),),
        in_specs=[
            pl.BlockSpec((_B, _EMB), lambda i: (0, 0)),
            pl.BlockSpec((_VT, _EMB), lambda i: (i, 0)),
            pl.BlockSpec((1, _VT), lambda i: (0, i)),
        ],
        out_specs=pl.BlockSpec((_B, _VT), lambda i: (0, i)),
        out_shape=jax.ShapeDtypeStruct((_B, _VOCAB), jnp.float32),
    )(pooled, fc_w, fc_b2)


def kernel(x, emb_table, fc_w, fc_b):
    xi = x.astype(jnp.int32).reshape(_B * _L // _CH, _CH)
    pooled = _pool(xi, emb_table)
    return _fc(pooled, fc_w, fc_b.reshape(1, _VOCAB))


# speedup vs baseline: 1.4410x; 1.0315x over previous
import functools

import jax
import jax.numpy as jnp
from jax import lax
from jax.experimental import pallas as pl
from jax.experimental.pallas import tpu as pltpu
from jax.experimental.pallas import tpu_sc as plsc

_VOCAB = 100000
_EMB = 32
_B = 1024
_L = 200

_NC = 2
_NS = 16
_NW = _NC * _NS
_BPW = _B // _NW
_CH = 100
_CPW = _BPW * _L // _CH
_LANES = 16

_mesh = plsc.VectorSubcoreMesh(core_axis_name="c", subcore_axis_name="s")


@functools.partial(
    pl.kernel,
    mesh=_mesh,
    out_type=jax.ShapeDtypeStruct((_B, _EMB), jnp.float32),
    scratch_types=[
        pltpu.VMEM((_CPW, _CH), jnp.int32),
        pltpu.VMEM((_CH, _EMB), jnp.float32),
        pltpu.VMEM((_CH, _EMB), jnp.float32),
        pltpu.VMEM((_BPW, _EMB), jnp.float32),
        pltpu.SemaphoreType.DMA,
        pltpu.SemaphoreType.DMA,
    ],
    compiler_params=pltpu.CompilerParams(use_tc_tiling_on_sc=False),
)
def _pool(x_hbm, table_hbm, out_hbm, idx_v, rows0_v, rows1_v, pooled_v,
          sem0, sem1):
    wid = lax.axis_index("s") * _NC + lax.axis_index("c")
    pltpu.sync_copy(x_hbm.at[pl.ds(wid * _CPW, _CPW)], idx_v)
    inv_l = jnp.float32(1.0 / _L)

    def _accum(buf):
        def body(l, c):
            a0, a1 = c
            return (a0 + buf[l, pl.ds(0, _LANES)],
                    a1 + buf[l, pl.ds(_LANES, _LANES)])
        z = jnp.zeros((_LANES,), jnp.float32)
        return lax.fori_loop(0, _CH, body, (z, z))

    def row_body(i, carry):
        j0 = 2 * i
        pltpu.async_copy(table_hbm.at[idx_v.at[j0]], rows0_v, sem0)
        pltpu.async_copy(table_hbm.at[idx_v.at[j0 + 1]], rows1_v, sem1)
        pltpu.make_async_copy(table_hbm.at[idx_v.at[j0]], rows0_v, sem0).wait()
        a0, a1 = _accum(rows0_v)
        pltpu.make_async_copy(table_hbm.at[idx_v.at[j0 + 1]], rows1_v,
                              sem1).wait()
        b0, b1 = _accum(rows1_v)
        pooled_v[i, pl.ds(0, _LANES)] = (a0 + b0) * inv_l
        pooled_v[i, pl.ds(_LANES, _LANES)] = (a1 + b1) * inv_l
        return carry

    lax.fori_loop(0, _BPW, row_body, 0)
    pltpu.sync_copy(pooled_v, out_hbm.at[pl.ds(wid * _BPW, _BPW)])


_VT = 2048


def _fc_body(p_ref, w_ref, b_ref, o_ref):
    o_ref[...] = lax.dot_general(
        p_ref[...], w_ref[...],
        dimension_numbers=(((1,), (1,)), ((), ())),
        preferred_element_type=jnp.float32,
    ) + b_ref[...]


def _fc(pooled, fc_w, fc_b2):
    return pl.pallas_call(
        _fc_body,
        grid=(pl.cdiv(_VOCAB, _VT),),
        in_specs=[
            pl.BlockSpec((_B, _EMB), lambda i: (0, 0)),
            pl.BlockSpec((_VT, _EMB), lambda i: (i, 0)),
            pl.BlockSpec((1, _VT), lambda i: (0, i)),
        ],
        out_specs=pl.BlockSpec((_B, _VT), lambda i: (0, i)),
        out_shape=jax.ShapeDtypeStruct((_B, _VOCAB), jnp.float32),
    )(pooled, fc_w, fc_b2)


def kernel(x, emb_table, fc_w, fc_b):
    xi = x.astype(jnp.int32).reshape(_B * _L // _CH, _CH)
    pooled = _pool(xi, emb_table)
    return _fc(pooled, fc_w, fc_b.reshape(1, _VOCAB))


# speedup vs baseline: 1.4497x; 1.0060x over previous
import functools

import jax
import jax.numpy as jnp
from jax import lax
from jax.experimental import pallas as pl
from jax.experimental.pallas import tpu as pltpu
from jax.experimental.pallas import tpu_sc as plsc

_VOCAB = 100000
_EMB = 32
_B = 1024
_L = 200

_NC = 2
_NS = 16
_NW = _NC * _NS
_BPW = _B // _NW
_CH = 100
_CPW = _BPW * _L // _CH
_LANES = 16

_mesh = plsc.VectorSubcoreMesh(core_axis_name="c", subcore_axis_name="s")


@functools.partial(
    pl.kernel,
    mesh=_mesh,
    out_type=jax.ShapeDtypeStruct((_B, _EMB), jnp.float32),
    scratch_types=[
        pltpu.VMEM((_CPW, _CH), jnp.int32),
        pltpu.VMEM((_CH, _EMB), jnp.float32),
        pltpu.VMEM((_CH, _EMB), jnp.float32),
        pltpu.VMEM((_BPW, _EMB), jnp.float32),
        pltpu.SemaphoreType.DMA,
        pltpu.SemaphoreType.DMA,
    ],
    compiler_params=pltpu.CompilerParams(use_tc_tiling_on_sc=False),
)
def _pool(x_hbm, table_hbm, out_hbm, idx_v, rows0_v, rows1_v, pooled_v,
          sem0, sem1):
    wid = lax.axis_index("s") * _NC + lax.axis_index("c")
    pltpu.sync_copy(x_hbm.at[pl.ds(wid * _CPW, _CPW)], idx_v)
    inv_l = jnp.float32(1.0 / _L)

    def _accum(buf):
        def body(l, c):
            a0, a1 = c
            return (a0 + buf[l, pl.ds(0, _LANES)],
                    a1 + buf[l, pl.ds(_LANES, _LANES)])
        z = jnp.zeros((_LANES,), jnp.float32)
        return lax.fori_loop(0, _CH, body, (z, z))

    def row_body(i, carry):
        j0 = 2 * i
        pltpu.async_copy(table_hbm.at[idx_v.at[j0]], rows0_v, sem0)
        pltpu.async_copy(table_hbm.at[idx_v.at[j0 + 1]], rows1_v, sem1)
        pltpu.make_async_copy(table_hbm.at[idx_v.at[j0]], rows0_v, sem0).wait()
        a0, a1 = _accum(rows0_v)
        pltpu.make_async_copy(table_hbm.at[idx_v.at[j0 + 1]], rows1_v,
                              sem1).wait()
        b0, b1 = _accum(rows1_v)
        pooled_v[i, pl.ds(0, _LANES)] = (a0 + b0) * inv_l
        pooled_v[i, pl.ds(_LANES, _LANES)] = (a1 + b1) * inv_l
        return carry

    lax.fori_loop(0, _BPW, row_body, 0)
    pltpu.sync_copy(pooled_v, out_hbm.at[pl.ds(wid * _BPW, _BPW)])


_VT = 4096


def _fc_body(p_ref, w_ref, b_ref, o_ref):
    o_ref[...] = lax.dot_general(
        p_ref[...], w_ref[...],
        dimension_numbers=(((1,), (1,)), ((), ())),
        preferred_element_type=jnp.float32,
    ) + b_ref[...]


def _fc(pooled, fc_w, fc_b2):
    return pl.pallas_call(
        _fc_body,
        grid=(pl.cdiv(_VOCAB, _VT),),
        in_specs=[
            pl.BlockSpec((_B, _EMB), lambda i: (0, 0)),
            pl.BlockSpec((_VT, _EMB), lambda i: (i, 0)),
            pl.BlockSpec((1, _VT), lambda i: (0, i)),
        ],
        out_specs=pl.BlockSpec((_B, _VT), lambda i: (0, i)),
        out_shape=jax.ShapeDtypeStruct((_B, _VOCAB), jnp.float32),
    )(pooled, fc_w, fc_b2)


def kernel(x, emb_table, fc_w, fc_b):
    xi = x.astype(jnp.int32).reshape(_B * _L // _CH, _CH)
    pooled = _pool(xi, emb_table)
    return _fc(pooled, fc_w, fc_b.reshape(1, _VOCAB))


# speedup vs baseline: 1.5026x; 1.0365x over previous
import functools

import jax
import jax.numpy as jnp
from jax import lax
from jax.experimental import pallas as pl
from jax.experimental.pallas import tpu as pltpu
from jax.experimental.pallas import tpu_sc as plsc

_VOCAB = 100000
_EMB = 32
_B = 1024
_L = 200

_NC = 2
_NS = 16
_NW = _NC * _NS
_BPW = _B // _NW
_CH = 100
_CPW = _BPW * _L // _CH
_LANES = 16

_mesh = plsc.VectorSubcoreMesh(core_axis_name="c", subcore_axis_name="s")


@functools.partial(
    pl.kernel,
    mesh=_mesh,
    out_type=jax.ShapeDtypeStruct((_B, _EMB), jnp.float32),
    scratch_types=[
        pltpu.VMEM((_CPW, _CH), jnp.int32),
        pltpu.VMEM((_CH, _EMB), jnp.float32),
        pltpu.VMEM((_CH, _EMB), jnp.float32),
        pltpu.VMEM((_CH, _EMB), jnp.float32),
        pltpu.VMEM((_CH, _EMB), jnp.float32),
        pltpu.VMEM((_BPW, _EMB), jnp.float32),
        pltpu.SemaphoreType.DMA,
        pltpu.SemaphoreType.DMA,
        pltpu.SemaphoreType.DMA,
        pltpu.SemaphoreType.DMA,
    ],
    compiler_params=pltpu.CompilerParams(use_tc_tiling_on_sc=False),
)
def _pool(x_hbm, table_hbm, out_hbm, idx_v, rows0_v, rows1_v, rows2_v,
          rows3_v, pooled_v, sem0, sem1, sem2, sem3):
    wid = lax.axis_index("s") * _NC + lax.axis_index("c")
    pltpu.sync_copy(x_hbm.at[pl.ds(wid * _CPW, _CPW)], idx_v)
    inv_l = jnp.float32(1.0 / _L)
    bufs = (rows0_v, rows1_v, rows2_v, rows3_v)
    sems = (sem0, sem1, sem2, sem3)

    def _fire(j, buf, sem):
        pltpu.async_copy(table_hbm.at[idx_v.at[j]], buf, sem)

    def _wait(j, buf, sem):
        pltpu.make_async_copy(table_hbm.at[idx_v.at[j]], buf, sem).wait()

    def _accum(buf):
        def body(l, c):
            b = l * 4
            new = []
            for u in range(4):
                new.append(c[2 * u] + buf[b + u, pl.ds(0, _LANES)])
                new.append(c[2 * u + 1] + buf[b + u, pl.ds(_LANES, _LANES)])
            return tuple(new)
        z = jnp.zeros((_LANES,), jnp.float32)
        c = lax.fori_loop(0, _CH // 4, body, (z,) * 8)
        return ((c[0] + c[2]) + (c[4] + c[6]),
                (c[1] + c[3]) + (c[5] + c[7]))

    _fire(0, bufs[0], sems[0])
    _fire(1, bufs[1], sems[1])
    _fire(2, bufs[2], sems[2])
    _fire(3, bufs[3], sems[3])

    def pair_body(k, carry):
        j = 4 * k
        for h in range(2):
            jc = j + 2 * h
            b0, b1 = bufs[2 * h], bufs[2 * h + 1]
            s0, s1 = sems[2 * h], sems[2 * h + 1]
            _wait(jc, b0, s0)
            a_lo, a_hi = _accum(b0)
            _wait(jc + 1, b1, s1)
            c_lo, c_hi = _accum(b1)

            @pl.when(k < _BPW // 2 - 1)
            def _prefetch():
                _fire(jc + 4, b0, s0)
                _fire(jc + 5, b1, s1)

            pooled_v[2 * k + h, pl.ds(0, _LANES)] = (a_lo + c_lo) * inv_l
            pooled_v[2 * k + h, pl.ds(_LANES, _LANES)] = (a_hi + c_hi) * inv_l
        return carry

    lax.fori_loop(0, _BPW // 2, pair_body, 0)
    pltpu.sync_copy(pooled_v, out_hbm.at[pl.ds(wid * _BPW, _BPW)])


_VT = 4096


def _fc_body(p_ref, w_ref, b_ref, o_ref):
    o_ref[...] = lax.dot_general(
        p_ref[...], w_ref[...],
        dimension_numbers=(((1,), (1,)), ((), ())),
        preferred_element_type=jnp.float32,
    ) + b_ref[...]


def _fc(pooled, fc_w, fc_b2):
    return pl.pallas_call(
        _fc_body,
        grid=(pl.cdiv(_VOCAB, _VT),),
        in_specs=[
            pl.BlockSpec((_B, _EMB), lambda i: (0, 0)),
            pl.BlockSpec((_VT, _EMB), lambda i: (i, 0)),
            pl.BlockSpec((1, _VT), lambda i: (0, i)),
        ],
        out_specs=pl.BlockSpec((_B, _VT), lambda i: (0, i)),
        out_shape=jax.ShapeDtypeStruct((_B, _VOCAB), jnp.float32),
    )(pooled, fc_w, fc_b2)


def kernel(x, emb_table, fc_w, fc_b):
    xi = x.astype(jnp.int32).reshape(_B * _L // _CH, _CH)
    pooled = _pool(xi, emb_table)
    return _fc(pooled, fc_w, fc_b.reshape(1, _VOCAB))
